# baseline (speedup 1.0000x reference)
import functools

import jax
import jax.numpy as jnp
from jax import lax
from jax.experimental import pallas as pl
from jax.experimental.pallas import tpu as pltpu
from jax.experimental.pallas import tpu_sc as plsc

B = 16
N0 = 2048
S1 = 512
S2 = 128
K1 = 32
K2 = 64
R1SQ = 0.2 * 0.2
R2SQ = 0.4 * 0.4
EPS = 1e-5

NC = 2
NS = 16
LANES = 16
NTILES = NC * NS


def _fps_body(pts_ref, cen_ref, pp_ref, scr_ref):
    x = pts_ref[:, 0, :]
    y = pts_ref[:, 1, :]
    z = pts_ref[:, 2, :]
    iota_n = lax.broadcasted_iota(jnp.int32, (B, N0), 1)

    def step(t, carry):
        dist, idxv = carry
        oh = (iota_n == idxv).astype(jnp.float32)
        cx = jnp.sum(x * oh, axis=1, keepdims=True)
        cy = jnp.sum(y * oh, axis=1, keepdims=True)
        cz = jnp.sum(z * oh, axis=1, keepdims=True)
        scr_ref[0, pl.ds(t, 1), :] = cx.reshape(1, B)
        scr_ref[1, pl.ds(t, 1), :] = cy.reshape(1, B)
        scr_ref[2, pl.ds(t, 1), :] = cz.reshape(1, B)
        dx = x - cx
        dy = y - cy
        dz = z - cz
        d = (dx * dx + dy * dy) + dz * dz
        dist = jnp.minimum(dist, d)
        mx = jnp.max(dist, axis=1, keepdims=True)
        idxv = jnp.min(jnp.where(dist == mx, iota_n, N0), axis=1, keepdims=True)
        return dist, idxv

    dist0 = jnp.full((B, N0), 1e10, dtype=jnp.float32)
    idx0 = jnp.zeros((B, 1), dtype=jnp.int32)
    lax.fori_loop(0, S1, step, (dist0, idx0))
    for c in range(3):
        cen_ref[pl.ds(c * B, B), :] = jnp.transpose(scr_ref[c])
        pp_ref[pl.ds(c * B, B), :] = pts_ref[:, c, :]


def _fps(points):
    return pl.pallas_call(
        _fps_body,
        out_shape=[
            jax.ShapeDtypeStruct((3 * B, S1), jnp.float32),
            jax.ShapeDtypeStruct((3 * B, N0), jnp.float32),
        ],
        scratch_shapes=[pltpu.VMEM((3, S1, B), jnp.float32)],
    )(points)


def _make_ballquery(n, s, k, r2, want_idx):
    spt = (B * s) // NTILES
    tiles_per_b = s // spt
    nchunks = n // LANES
    bufsz = k + LANES

    def body(*refs):
        pts_hbm, cen_hbm = refs[0], refs[1]
        if want_idx:
            outs = refs[2:6]
            vpx, vpy, vpz, vcx, vcy, vcz, vbuf, vrel, vidx, sem = refs[6:]
        else:
            outs = refs[2:5]
            vpx, vpy, vpz, vcx, vcy, vcz, vbuf, vrel, sem = refs[5:]
            vidx = None
        w = lax.axis_index("s") * NC + lax.axis_index("c")
        b = w // tiles_per_b
        s0 = (w % tiles_per_b) * spt
        pltpu.sync_copy(pts_hbm.at[pl.ds(b * n, n)], vpx)
        pltpu.sync_copy(pts_hbm.at[pl.ds((B + b) * n, n)], vpy)
        pltpu.sync_copy(pts_hbm.at[pl.ds((2 * B + b) * n, n)], vpz)
        cbase = b * S1 + s0
        pltpu.sync_copy(cen_hbm.at[pl.ds(cbase, spt)], vcx)
        pltpu.sync_copy(cen_hbm.at[pl.ds(cbase + B * S1, spt)], vcy)
        pltpu.sync_copy(cen_hbm.at[pl.ds(cbase + 2 * B * S1, spt)], vcz)
        lane = lax.iota(jnp.int32, LANES)
        zero = lane * 0

        def per_s(si, _):
            sv = zero + si
            cx = plsc.load_gather(vcx, [sv])
            cy = plsc.load_gather(vcy, [sv])
            cz = plsc.load_gather(vcz, [sv])

            def cond(c):
                i, cnt = c
                return jnp.logical_and(i < nchunks, cnt < k)

            def chunk(c):
                i, cnt = c
                ii = i * LANES
                dx = vpx[pl.ds(ii, LANES)] - cx
                dy = vpy[pl.ds(ii, LANES)] - cy
                dz = vpz[pl.ds(ii, LANES)] - cz
                d2 = (dx * dx + dy * dy) + dz * dz
                m = d2 < r2
                mi = m.astype(jnp.int32)
                incl = plsc.cumsum(mi)
                pos = jnp.minimum(cnt, k) + (incl - mi)
                plsc.store_scatter(vbuf, [pos], ii + lane, mask=m)
                return i + 1, cnt + incl[LANES - 1]

            _, cnt = lax.while_loop(cond, chunk, (0, 0))
            valid = jnp.minimum(cnt, k)
            for j in range(k // LANES):
                p = j * LANES + lane
                pos = jnp.where(p < valid, p, 0)
                gi = plsc.load_gather(vbuf, [pos])
                vrel[0, si, pl.ds(j * LANES, LANES)] = (
                    plsc.load_gather(vpx, [gi]) - cx)
                vrel[1, si, pl.ds(j * LANES, LANES)] = (
                    plsc.load_gather(vpy, [gi]) - cy)
                vrel[2, si, pl.ds(j * LANES, LANES)] = (
                    plsc.load_gather(vpz, [gi]) - cz)
                if want_idx:
                    vidx[si, pl.ds(j * LANES, LANES)] = gi
            return _

        lax.fori_loop(0, spt, per_s, 0)
        pltpu.sync_copy(vrel.at[0], outs[0].at[b, pl.ds(s0, spt), :])
        pltpu.sync_copy(vrel.at[1], outs[1].at[b, pl.ds(s0, spt), :])
        pltpu.sync_copy(vrel.at[2], outs[2].at[b, pl.ds(s0, spt), :])
        if want_idx:
            pltpu.sync_copy(vidx, outs[3].at[b, pl.ds(s0, spt), :])

    mesh = plsc.VectorSubcoreMesh(core_axis_name="c", subcore_axis_name="s")
    out_type = [jax.ShapeDtypeStruct((B, s, k), jnp.float32)] * 3
    if want_idx:
        out_type.append(jax.ShapeDtypeStruct((B, s, k), jnp.int32))
    scratch = [
        pltpu.VMEM((n,), jnp.float32),
        pltpu.VMEM((n,), jnp.float32),
        pltpu.VMEM((n,), jnp.float32),
        pltpu.VMEM((spt,), jnp.float32),
        pltpu.VMEM((spt,), jnp.float32),
        pltpu.VMEM((spt,), jnp.float32),
        pltpu.VMEM((bufsz,), jnp.int32),
        pltpu.VMEM((3, spt, k), jnp.float32),
    ]
    if want_idx:
        scratch.append(pltpu.VMEM((spt, k), jnp.int32))
    scratch.append(pltpu.SemaphoreType.DMA)
    return pl.kernel(body, out_type=tuple(out_type), mesh=mesh,
                     scratch_types=scratch,
                     compiler_params=pltpu.CompilerParams(
                         needs_layout_passes=False))


_SA1_CH = 4


def _bn_relu(y, gamma, beta):
    return jnp.maximum((y / jnp.sqrt(1.0 + EPS)) * gamma + beta, 0.0)


def _sa1_body(rx_ref, ry_ref, rz_ref, w1, g1, b1, w2, g2, b2, w3, g3, b3,
              out_ref):
    sc = S1 // _SA1_CH
    m = sc * K1
    x = jnp.concatenate([rx_ref[0].reshape(1, m), ry_ref[0].reshape(1, m),
                         rz_ref[0].reshape(1, m)], axis=0)
    h = _bn_relu(jnp.dot(w1[...], x, preferred_element_type=jnp.float32),
                 g1[...], b1[...])
    h = _bn_relu(jnp.dot(w2[...], h, preferred_element_type=jnp.float32),
                 g2[...], b2[...])
    h = _bn_relu(jnp.dot(w3[...], h, preferred_element_type=jnp.float32),
                 g3[...], b3[...])
    out_ref[0] = jnp.max(h.reshape(128, sc, K1), axis=2)


def _sa1_mlp(r1x, r1y, r1z, ws):
    sc = S1 // _SA1_CH
    (w1, g1, b1), (w2, g2, b2), (w3, g3, b3) = ws
    cvec = lambda c: pl.BlockSpec((c, 1), lambda b, ch: (0, 0))
    return pl.pallas_call(
        _sa1_body,
        grid=(B, _SA1_CH),
        in_specs=[
            pl.BlockSpec((1, sc, K1), lambda b, c: (b, c, 0)),
            pl.BlockSpec((1, sc, K1), lambda b, c: (b, c, 0)),
            pl.BlockSpec((1, sc, K1), lambda b, c: (b, c, 0)),
            pl.BlockSpec((64, 3), lambda b, c: (0, 0)), cvec(64), cvec(64),
            pl.BlockSpec((64, 64), lambda b, c: (0, 0)), cvec(64), cvec(64),
            pl.BlockSpec((128, 64), lambda b, c: (0, 0)), cvec(128), cvec(128),
        ],
        out_specs=pl.BlockSpec((1, 128, sc), lambda b, c: (b, 0, c)),
        out_shape=jax.ShapeDtypeStruct((B, 128, S1), jnp.float32),
    )(r1x, r1y, r1z, w1, g1, b1, w2, g2, b2, w3, g3, b3)


_SA2_CH = 8
_SA2_CW = (S2 * K2) // _SA2_CH
_SA2_SPC = _SA2_CW // K2


def _sa2_body(idx_ref, rx_ref, ry_ref, rz_ref, f1_ref,
              w1, g1, b1, w2, g2, b2, w3, g3, b3,
              lw1, lg1, lb1, lw2, lg2, lb2, lw3, lg3, lb3,
              feats_ref, ami_ref):
    f1 = f1_ref[0]
    iota_n = lax.broadcasted_iota(jnp.int32, (S1, _SA2_CW), 0)

    pooled = []
    for c in range(_SA2_CH):
        rows = pl.ds(c * _SA2_SPC, _SA2_SPC)
        idr = idx_ref[0, rows, :].reshape(1, _SA2_CW)
        e = (iota_n == idr).astype(jnp.float32)
        g = jnp.dot(f1, e, preferred_element_type=jnp.float32)
        x = jnp.concatenate(
            [rx_ref[0, rows, :].reshape(1, _SA2_CW),
             ry_ref[0, rows, :].reshape(1, _SA2_CW),
             rz_ref[0, rows, :].reshape(1, _SA2_CW), g], axis=0)
        h = _bn_relu(jnp.dot(w1[...], x, preferred_element_type=jnp.float32),
                     g1[...], b1[...])
        h = _bn_relu(jnp.dot(w2[...], h, preferred_element_type=jnp.float32),
                     g2[...], b2[...])
        h = _bn_relu(jnp.dot(w3[...], h, preferred_element_type=jnp.float32),
                     g3[...], b3[...])
        pooled.append(jnp.max(h.reshape(256, _SA2_SPC, K2), axis=2))

    f2 = jnp.concatenate(pooled, axis=1)
    h = _bn_relu(jnp.dot(lw1[...], f2, preferred_element_type=jnp.float32),
                 lg1[...], lb1[...])
    h = _bn_relu(jnp.dot(lw2[...], h, preferred_element_type=jnp.float32),
                 lg2[...], lb2[...])
    h = _bn_relu(jnp.dot(lw3[...], h, preferred_element_type=jnp.float32),
                 lg3[...], lb3[...])
    mx = jnp.max(h, axis=1)
    feats_ref[0, 0] = mx
    iota_s = lax.broadcasted_iota(jnp.int32, (1024, S2), 1)
    ami = jnp.min(jnp.where(h == mx[:, None], iota_s, S2), axis=1)
    ami_ref[0, 0] = ami.astype(jnp.int32)


def _sa2_local(idx2, r2x, r2y, r2z, feat1, ws2, wsl):
    (w1, g1, b1), (w2, g2, b2), (w3, g3, b3) = ws2
    (lw1, lg1, lb1), (lw2, lg2, lb2), (lw3, lg3, lb3) = wsl
    full = lambda *shape: pl.BlockSpec(shape, lambda b: tuple(0 for _ in shape))
    return pl.pallas_call(
        _sa2_body,
        grid=(B,),
        in_specs=[
            pl.BlockSpec((1, S2, K2), lambda b: (b, 0, 0)),
            pl.BlockSpec((1, S2, K2), lambda b: (b, 0, 0)),
            pl.BlockSpec((1, S2, K2), lambda b: (b, 0, 0)),
            pl.BlockSpec((1, S2, K2), lambda b: (b, 0, 0)),
            pl.BlockSpec((1, 128, S1), lambda b: (b, 0, 0)),
            full(128, 131), full(128, 1), full(128, 1),
            full(128, 128), full(128, 1), full(128, 1),
            full(256, 128), full(256, 1), full(256, 1),
            full(256, 256), full(256, 1), full(256, 1),
            full(512, 256), full(512, 1), full(512, 1),
            full(1024, 512), full(1024, 1), full(1024, 1),
        ],
        out_specs=[
            pl.BlockSpec((1, 1, 1024), lambda b: (b, 0, 0)),
            pl.BlockSpec((1, 1, 1024), lambda b: (b, 0, 0)),
        ],
        out_shape=[
            jax.ShapeDtypeStruct((B, 1, 1024), jnp.float32),
            jax.ShapeDtypeStruct((B, 1, 1024), jnp.int32),
        ],
    )(idx2, r2x, r2y, r2z, feat1, w1, g1, b1, w2, g2, b2, w3, g3, b3,
      lw1, lg1, lb1, lw2, lg2, lb2, lw3, lg3, lb3)


def _prep(layers):
    return [(w, gamma[:, None], beta[:, None]) for w, gamma, beta in layers]


def kernel(points, params):
    cen_pm, pts_pm = _fps(points)
    points_f = pts_pm.reshape(-1)
    centers_f = cen_pm.reshape(-1)

    bq1 = _make_ballquery(N0, S1, K1, R1SQ, want_idx=False)
    r1x, r1y, r1z = bq1(points_f, centers_f)

    sa1 = _prep(params["sa1"])
    feat1 = _sa1_mlp(r1x, r1y, r1z, sa1)

    bq2 = _make_ballquery(S1, S2, K2, R2SQ, want_idx=True)
    r2x, r2y, r2z, idx2 = bq2(centers_f, centers_f)

    ws2 = _prep(params["sa2"])
    wsl = _prep(params["local"])
    feats, ami = _sa2_local(idx2, r2x, r2y, r2z, feat1, ws2, wsl)
    return {"feats": feats[:, 0], "max_indices": ami[:, 0]}


# speedup vs baseline: 1.1455x; 1.1455x over previous
import functools

import jax
import jax.numpy as jnp
from jax import lax
from jax.experimental import pallas as pl
from jax.experimental.pallas import tpu as pltpu
from jax.experimental.pallas import tpu_sc as plsc

B = 16
N0 = 2048
S1 = 512
S2 = 128
K1 = 32
K2 = 64
R1SQ = 0.2 * 0.2
R2SQ = 0.4 * 0.4
EPS = 1e-5

NC = 2
NS = 16
LANES = 16
NTILES = NC * NS


def _fps_body(pts_ref, cen_ref, pp_ref, scr_ref):
    x = pts_ref[:, 0, :]
    y = pts_ref[:, 1, :]
    z = pts_ref[:, 2, :]
    iota_n = lax.broadcasted_iota(jnp.int32, (B, N0), 1)

    def step(t, carry):
        dist, idxv = carry
        oh = (iota_n == idxv).astype(jnp.float32)
        cx = jnp.sum(x * oh, axis=1, keepdims=True)
        cy = jnp.sum(y * oh, axis=1, keepdims=True)
        cz = jnp.sum(z * oh, axis=1, keepdims=True)
        scr_ref[0, pl.ds(t, 1), :] = cx.reshape(1, B)
        scr_ref[1, pl.ds(t, 1), :] = cy.reshape(1, B)
        scr_ref[2, pl.ds(t, 1), :] = cz.reshape(1, B)
        dx = x - cx
        dy = y - cy
        dz = z - cz
        d = (dx * dx + dy * dy) + dz * dz
        dist = jnp.minimum(dist, d)
        mx = jnp.max(dist, axis=1, keepdims=True)
        idxv = jnp.min(jnp.where(dist == mx, iota_n, N0), axis=1, keepdims=True)
        return dist, idxv

    dist0 = jnp.full((B, N0), 1e10, dtype=jnp.float32)
    idx0 = jnp.zeros((B, 1), dtype=jnp.int32)
    lax.fori_loop(0, S1, step, (dist0, idx0))
    for c in range(3):
        cen_ref[pl.ds(c * B, B), :] = jnp.transpose(scr_ref[c])
        pp_ref[pl.ds(c * B, B), :] = pts_ref[:, c, :]


def _fps(points):
    return pl.pallas_call(
        _fps_body,
        out_shape=[
            jax.ShapeDtypeStruct((3 * B, S1), jnp.float32),
            jax.ShapeDtypeStruct((3 * B, N0), jnp.float32),
        ],
        scratch_shapes=[pltpu.VMEM((3, S1, B), jnp.float32)],
    )(points)


def _make_ballquery(n, s, k, r2, want_idx):
    spt = (B * s) // NTILES
    tiles_per_b = s // spt
    nchunks = n // LANES
    UB = 16
    nblocks = nchunks // UB
    bufsz = k + LANES

    def body(*refs):
        pts_hbm, cen_hbm = refs[0], refs[1]
        if want_idx:
            outs = refs[2:6]
            vpx, vpy, vpz, vcx, vcy, vcz, vbuf, vrel, vidx, sem = refs[6:]
        else:
            outs = refs[2:5]
            vpx, vpy, vpz, vcx, vcy, vcz, vbuf, vrel, sem = refs[5:]
            vidx = None
        w = lax.axis_index("s") * NC + lax.axis_index("c")
        b = w // tiles_per_b
        s0 = (w % tiles_per_b) * spt
        pltpu.sync_copy(pts_hbm.at[pl.ds(b * n, n)], vpx)
        pltpu.sync_copy(pts_hbm.at[pl.ds((B + b) * n, n)], vpy)
        pltpu.sync_copy(pts_hbm.at[pl.ds((2 * B + b) * n, n)], vpz)
        cbase = b * S1 + s0
        pltpu.sync_copy(cen_hbm.at[pl.ds(cbase, spt)], vcx)
        pltpu.sync_copy(cen_hbm.at[pl.ds(cbase + B * S1, spt)], vcy)
        pltpu.sync_copy(cen_hbm.at[pl.ds(cbase + 2 * B * S1, spt)], vcz)
        lane = lax.iota(jnp.int32, LANES)
        zero = lane * 0

        def per_s(si, _):
            sv = zero + si
            cx = plsc.load_gather(vcx, [sv])
            cy = plsc.load_gather(vcy, [sv])
            cz = plsc.load_gather(vcz, [sv])

            def cond(c):
                i, cnt = c
                return jnp.logical_and(i < nblocks, cnt < k)

            def block(c):
                i, cnt = c
                base = i * (UB * LANES)
                for u in range(UB):
                    ii = base + u * LANES
                    dx = vpx[pl.ds(ii, LANES)] - cx
                    dy = vpy[pl.ds(ii, LANES)] - cy
                    dz = vpz[pl.ds(ii, LANES)] - cz
                    d2 = (dx * dx + dy * dy) + dz * dz
                    m = d2 < r2
                    mi = m.astype(jnp.int32)
                    incl = plsc.cumsum(mi)
                    pos = jnp.minimum(cnt, k) + (incl - mi)
                    plsc.store_scatter(vbuf, [pos], ii + lane, mask=m)
                    cnt = cnt + incl[LANES - 1]
                return i + 1, cnt

            _, cnt = lax.while_loop(cond, block, (0, 0))
            valid = jnp.minimum(cnt, k)
            for j in range(k // LANES):
                p = j * LANES + lane
                pos = jnp.where(p < valid, p, 0)
                gi = plsc.load_gather(vbuf, [pos])
                vrel[0, si, pl.ds(j * LANES, LANES)] = (
                    plsc.load_gather(vpx, [gi]) - cx)
                vrel[1, si, pl.ds(j * LANES, LANES)] = (
                    plsc.load_gather(vpy, [gi]) - cy)
                vrel[2, si, pl.ds(j * LANES, LANES)] = (
                    plsc.load_gather(vpz, [gi]) - cz)
                if want_idx:
                    vidx[si, pl.ds(j * LANES, LANES)] = gi
            return _

        lax.fori_loop(0, spt, per_s, 0)
        pltpu.sync_copy(vrel.at[0], outs[0].at[b, pl.ds(s0, spt), :])
        pltpu.sync_copy(vrel.at[1], outs[1].at[b, pl.ds(s0, spt), :])
        pltpu.sync_copy(vrel.at[2], outs[2].at[b, pl.ds(s0, spt), :])
        if want_idx:
            pltpu.sync_copy(vidx, outs[3].at[b, pl.ds(s0, spt), :])

    mesh = plsc.VectorSubcoreMesh(core_axis_name="c", subcore_axis_name="s")
    out_type = [jax.ShapeDtypeStruct((B, s, k), jnp.float32)] * 3
    if want_idx:
        out_type.append(jax.ShapeDtypeStruct((B, s, k), jnp.int32))
    scratch = [
        pltpu.VMEM((n,), jnp.float32),
        pltpu.VMEM((n,), jnp.float32),
        pltpu.VMEM((n,), jnp.float32),
        pltpu.VMEM((spt,), jnp.float32),
        pltpu.VMEM((spt,), jnp.float32),
        pltpu.VMEM((spt,), jnp.float32),
        pltpu.VMEM((bufsz,), jnp.int32),
        pltpu.VMEM((3, spt, k), jnp.float32),
    ]
    if want_idx:
        scratch.append(pltpu.VMEM((spt, k), jnp.int32))
    scratch.append(pltpu.SemaphoreType.DMA)
    return pl.kernel(body, out_type=tuple(out_type), mesh=mesh,
                     scratch_types=scratch,
                     compiler_params=pltpu.CompilerParams(
                         needs_layout_passes=False))


_SA1_CH = 4


def _bn_relu(y, gamma, beta):
    return jnp.maximum((y / jnp.sqrt(1.0 + EPS)) * gamma + beta, 0.0)


def _sa1_body(rx_ref, ry_ref, rz_ref, w1, g1, b1, w2, g2, b2, w3, g3, b3,
              out_ref):
    sc = S1 // _SA1_CH
    m = sc * K1
    x = jnp.concatenate([rx_ref[0].reshape(1, m), ry_ref[0].reshape(1, m),
                         rz_ref[0].reshape(1, m)], axis=0)
    h = _bn_relu(jnp.dot(w1[...], x, preferred_element_type=jnp.float32),
                 g1[...], b1[...])
    h = _bn_relu(jnp.dot(w2[...], h, preferred_element_type=jnp.float32),
                 g2[...], b2[...])
    h = _bn_relu(jnp.dot(w3[...], h, preferred_element_type=jnp.float32),
                 g3[...], b3[...])
    out_ref[0] = jnp.max(h.reshape(128, sc, K1), axis=2)


def _sa1_mlp(r1x, r1y, r1z, ws):
    sc = S1 // _SA1_CH
    (w1, g1, b1), (w2, g2, b2), (w3, g3, b3) = ws
    cvec = lambda c: pl.BlockSpec((c, 1), lambda b, ch: (0, 0))
    return pl.pallas_call(
        _sa1_body,
        grid=(B, _SA1_CH),
        in_specs=[
            pl.BlockSpec((1, sc, K1), lambda b, c: (b, c, 0)),
            pl.BlockSpec((1, sc, K1), lambda b, c: (b, c, 0)),
            pl.BlockSpec((1, sc, K1), lambda b, c: (b, c, 0)),
            pl.BlockSpec((64, 3), lambda b, c: (0, 0)), cvec(64), cvec(64),
            pl.BlockSpec((64, 64), lambda b, c: (0, 0)), cvec(64), cvec(64),
            pl.BlockSpec((128, 64), lambda b, c: (0, 0)), cvec(128), cvec(128),
        ],
        out_specs=pl.BlockSpec((1, 128, sc), lambda b, c: (b, 0, c)),
        out_shape=jax.ShapeDtypeStruct((B, 128, S1), jnp.float32),
    )(r1x, r1y, r1z, w1, g1, b1, w2, g2, b2, w3, g3, b3)


_SA2_CH = 8
_SA2_CW = (S2 * K2) // _SA2_CH
_SA2_SPC = _SA2_CW // K2


def _sa2_body(idx_ref, rx_ref, ry_ref, rz_ref, f1_ref,
              w1, g1, b1, w2, g2, b2, w3, g3, b3,
              lw1, lg1, lb1, lw2, lg2, lb2, lw3, lg3, lb3,
              feats_ref, ami_ref):
    f1 = f1_ref[0]
    iota_n = lax.broadcasted_iota(jnp.int32, (S1, _SA2_CW), 0)

    pooled = []
    for c in range(_SA2_CH):
        rows = pl.ds(c * _SA2_SPC, _SA2_SPC)
        idr = idx_ref[0, rows, :].reshape(1, _SA2_CW)
        e = (iota_n == idr).astype(jnp.float32)
        g = jnp.dot(f1, e, preferred_element_type=jnp.float32)
        x = jnp.concatenate(
            [rx_ref[0, rows, :].reshape(1, _SA2_CW),
             ry_ref[0, rows, :].reshape(1, _SA2_CW),
             rz_ref[0, rows, :].reshape(1, _SA2_CW), g], axis=0)
        h = _bn_relu(jnp.dot(w1[...], x, preferred_element_type=jnp.float32),
                     g1[...], b1[...])
        h = _bn_relu(jnp.dot(w2[...], h, preferred_element_type=jnp.float32),
                     g2[...], b2[...])
        h = _bn_relu(jnp.dot(w3[...], h, preferred_element_type=jnp.float32),
                     g3[...], b3[...])
        pooled.append(jnp.max(h.reshape(256, _SA2_SPC, K2), axis=2))

    f2 = jnp.concatenate(pooled, axis=1)
    h = _bn_relu(jnp.dot(lw1[...], f2, preferred_element_type=jnp.float32),
                 lg1[...], lb1[...])
    h = _bn_relu(jnp.dot(lw2[...], h, preferred_element_type=jnp.float32),
                 lg2[...], lb2[...])
    h = _bn_relu(jnp.dot(lw3[...], h, preferred_element_type=jnp.float32),
                 lg3[...], lb3[...])
    mx = jnp.max(h, axis=1)
    feats_ref[0, 0] = mx
    iota_s = lax.broadcasted_iota(jnp.int32, (1024, S2), 1)
    ami = jnp.min(jnp.where(h == mx[:, None], iota_s, S2), axis=1)
    ami_ref[0, 0] = ami.astype(jnp.int32)


def _sa2_local(idx2, r2x, r2y, r2z, feat1, ws2, wsl):
    (w1, g1, b1), (w2, g2, b2), (w3, g3, b3) = ws2
    (lw1, lg1, lb1), (lw2, lg2, lb2), (lw3, lg3, lb3) = wsl
    full = lambda *shape: pl.BlockSpec(shape, lambda b: tuple(0 for _ in shape))
    return pl.pallas_call(
        _sa2_body,
        grid=(B,),
        in_specs=[
            pl.BlockSpec((1, S2, K2), lambda b: (b, 0, 0)),
            pl.BlockSpec((1, S2, K2), lambda b: (b, 0, 0)),
            pl.BlockSpec((1, S2, K2), lambda b: (b, 0, 0)),
            pl.BlockSpec((1, S2, K2), lambda b: (b, 0, 0)),
            pl.BlockSpec((1, 128, S1), lambda b: (b, 0, 0)),
            full(128, 131), full(128, 1), full(128, 1),
            full(128, 128), full(128, 1), full(128, 1),
            full(256, 128), full(256, 1), full(256, 1),
            full(256, 256), full(256, 1), full(256, 1),
            full(512, 256), full(512, 1), full(512, 1),
            full(1024, 512), full(1024, 1), full(1024, 1),
        ],
        out_specs=[
            pl.BlockSpec((1, 1, 1024), lambda b: (b, 0, 0)),
            pl.BlockSpec((1, 1, 1024), lambda b: (b, 0, 0)),
        ],
        out_shape=[
            jax.ShapeDtypeStruct((B, 1, 1024), jnp.float32),
            jax.ShapeDtypeStruct((B, 1, 1024), jnp.int32),
        ],
    )(idx2, r2x, r2y, r2z, feat1, w1, g1, b1, w2, g2, b2, w3, g3, b3,
      lw1, lg1, lb1, lw2, lg2, lb2, lw3, lg3, lb3)


def _prep(layers):
    return [(w, gamma[:, None], beta[:, None]) for w, gamma, beta in layers]


def kernel(points, params):
    cen_pm, pts_pm = _fps(points)
    points_f = pts_pm.reshape(-1)
    centers_f = cen_pm.reshape(-1)

    bq1 = _make_ballquery(N0, S1, K1, R1SQ, want_idx=False)
    r1x, r1y, r1z = bq1(points_f, centers_f)

    sa1 = _prep(params["sa1"])
    feat1 = _sa1_mlp(r1x, r1y, r1z, sa1)

    bq2 = _make_ballquery(S1, S2, K2, R2SQ, want_idx=True)
    r2x, r2y, r2z, idx2 = bq2(centers_f, centers_f)

    ws2 = _prep(params["sa2"])
    wsl = _prep(params["local"])
    feats, ami = _sa2_local(idx2, r2x, r2y, r2z, feat1, ws2, wsl)
    return {"feats": feats[:, 0], "max_indices": ami[:, 0]}


# speedup vs baseline: 1.4676x; 1.2812x over previous
import functools

import jax
import jax.numpy as jnp
from jax import lax
from jax.experimental import pallas as pl
from jax.experimental.pallas import tpu as pltpu
from jax.experimental.pallas import tpu_sc as plsc

B = 16
N0 = 2048
S1 = 512
S2 = 128
K1 = 32
K2 = 64
R1SQ = 0.2 * 0.2
R2SQ = 0.4 * 0.4
EPS = 1e-5

NC = 2
NS = 16
LANES = 16
NTILES = NC * NS


def _fps_body(pts_ref, cen_ref, pp_ref, sx, sy, sz):
    x = pts_ref[:, 0, :]
    y = pts_ref[:, 1, :]
    z = pts_ref[:, 2, :]
    iota_n = lax.broadcasted_iota(jnp.int32, (B, N0), 1)

    def step(t, carry):
        dist, idxv = carry
        oh = (iota_n == idxv).astype(jnp.float32)
        cx = jnp.sum(x * oh, axis=1, keepdims=True)
        cy = jnp.sum(y * oh, axis=1, keepdims=True)
        cz = jnp.sum(z * oh, axis=1, keepdims=True)
        sx[pl.ds(t, 1), :] = cx.reshape(1, B)
        sy[pl.ds(t, 1), :] = cy.reshape(1, B)
        sz[pl.ds(t, 1), :] = cz.reshape(1, B)
        dx = x - cx
        dy = y - cy
        dz = z - cz
        d = (dx * dx + dy * dy) + dz * dz
        dist = jnp.minimum(dist, d)
        mx = jnp.max(dist, axis=1, keepdims=True)
        idxv = jnp.min(jnp.where(dist == mx, iota_n, N0), axis=1, keepdims=True)
        return dist, idxv

    dist0 = jnp.full((B, N0), 1e10, dtype=jnp.float32)
    idx0 = jnp.zeros((B, 1), dtype=jnp.int32)
    lax.fori_loop(0, S1, step, (dist0, idx0))
    for c, sref in enumerate((sx, sy, sz)):
        cen_ref[pl.ds(c * B, B), :] = jnp.transpose(sref[...])
        pp_ref[pl.ds(c * B, B), :] = pts_ref[:, c, :]


def _fps(points):
    return pl.pallas_call(
        _fps_body,
        out_shape=[
            jax.ShapeDtypeStruct((3 * B, S1), jnp.float32),
            jax.ShapeDtypeStruct((3 * B, N0), jnp.float32),
        ],
        scratch_shapes=[pltpu.VMEM((S1, B), jnp.float32),
                        pltpu.VMEM((S1, B), jnp.float32),
                        pltpu.VMEM((S1, B), jnp.float32)],
    )(points)


def _make_ballquery(n, s, k, r2, want_idx):
    spt = (B * s) // NTILES
    tiles_per_b = s // spt
    nchunks = n // LANES
    UB = 16
    nblocks = nchunks // UB
    bufsz = k + LANES

    def body(*refs):
        pts_hbm, cen_hbm = refs[0], refs[1]
        if want_idx:
            outs = refs[2:6]
            vpx, vpy, vpz, vcx, vcy, vcz, vbuf, vrel, vidx, sem = refs[6:]
        else:
            outs = refs[2:5]
            vpx, vpy, vpz, vcx, vcy, vcz, vbuf, vrel, sem = refs[5:]
            vidx = None
        w = lax.axis_index("s") * NC + lax.axis_index("c")
        b = w // tiles_per_b
        s0 = (w % tiles_per_b) * spt
        pltpu.sync_copy(pts_hbm.at[pl.ds(b * n, n)], vpx)
        pltpu.sync_copy(pts_hbm.at[pl.ds((B + b) * n, n)], vpy)
        pltpu.sync_copy(pts_hbm.at[pl.ds((2 * B + b) * n, n)], vpz)
        cbase = b * S1 + s0
        pltpu.sync_copy(cen_hbm.at[pl.ds(cbase, spt)], vcx)
        pltpu.sync_copy(cen_hbm.at[pl.ds(cbase + B * S1, spt)], vcy)
        pltpu.sync_copy(cen_hbm.at[pl.ds(cbase + 2 * B * S1, spt)], vcz)
        lane = lax.iota(jnp.int32, LANES)
        zero = lane * 0

        def per_s(si, _):
            sv = zero + si
            cx = plsc.load_gather(vcx, [sv])
            cy = plsc.load_gather(vcy, [sv])
            cz = plsc.load_gather(vcz, [sv])

            def cond(c):
                i, cnt = c
                return jnp.logical_and(i < nblocks, cnt < k)

            def block(c):
                i, cnt = c
                base = i * (UB * LANES)
                for u in range(UB):
                    ii = base + u * LANES
                    dx = vpx[pl.ds(ii, LANES)] - cx
                    dy = vpy[pl.ds(ii, LANES)] - cy
                    dz = vpz[pl.ds(ii, LANES)] - cz
                    d2 = (dx * dx + dy * dy) + dz * dz
                    m = d2 < r2
                    mi = m.astype(jnp.int32)
                    incl = plsc.cumsum(mi)
                    pos = jnp.minimum(cnt, k) + (incl - mi)
                    plsc.store_scatter(vbuf, [pos], ii + lane, mask=m)
                    cnt = cnt + incl[LANES - 1]
                return i + 1, cnt

            _, cnt = lax.while_loop(cond, block, (0, 0))
            valid = jnp.minimum(cnt, k)
            for j in range(k // LANES):
                p = j * LANES + lane
                pos = jnp.where(p < valid, p, 0)
                gi = plsc.load_gather(vbuf, [pos])
                vrel[0, si, pl.ds(j * LANES, LANES)] = (
                    plsc.load_gather(vpx, [gi]) - cx)
                vrel[1, si, pl.ds(j * LANES, LANES)] = (
                    plsc.load_gather(vpy, [gi]) - cy)
                vrel[2, si, pl.ds(j * LANES, LANES)] = (
                    plsc.load_gather(vpz, [gi]) - cz)
                if want_idx:
                    vidx[si, pl.ds(j * LANES, LANES)] = gi
            return _

        lax.fori_loop(0, spt, per_s, 0)
        pltpu.sync_copy(vrel.at[0], outs[0].at[b, pl.ds(s0, spt), :])
        pltpu.sync_copy(vrel.at[1], outs[1].at[b, pl.ds(s0, spt), :])
        pltpu.sync_copy(vrel.at[2], outs[2].at[b, pl.ds(s0, spt), :])
        if want_idx:
            pltpu.sync_copy(vidx, outs[3].at[b, pl.ds(s0, spt), :])

    mesh = plsc.VectorSubcoreMesh(core_axis_name="c", subcore_axis_name="s")
    out_type = [jax.ShapeDtypeStruct((B, s, k), jnp.float32)] * 3
    if want_idx:
        out_type.append(jax.ShapeDtypeStruct((B, s, k), jnp.int32))
    scratch = [
        pltpu.VMEM((n,), jnp.float32),
        pltpu.VMEM((n,), jnp.float32),
        pltpu.VMEM((n,), jnp.float32),
        pltpu.VMEM((spt,), jnp.float32),
        pltpu.VMEM((spt,), jnp.float32),
        pltpu.VMEM((spt,), jnp.float32),
        pltpu.VMEM((bufsz,), jnp.int32),
        pltpu.VMEM((3, spt, k), jnp.float32),
    ]
    if want_idx:
        scratch.append(pltpu.VMEM((spt, k), jnp.int32))
    scratch.append(pltpu.SemaphoreType.DMA)
    return pl.kernel(body, out_type=tuple(out_type), mesh=mesh,
                     scratch_types=scratch,
                     compiler_params=pltpu.CompilerParams(
                         needs_layout_passes=False))


_SA1_CH = 4


def _bn_relu(y, gamma, beta):
    return jnp.maximum((y / jnp.sqrt(1.0 + EPS)) * gamma + beta, 0.0)


def _sa1_body(rx_ref, ry_ref, rz_ref, w1, g1, b1, w2, g2, b2, w3, g3, b3,
              out_ref):
    sc = S1 // _SA1_CH
    m = sc * K1
    x = jnp.concatenate(
        [jnp.transpose(rx_ref[0]).reshape(1, m),
         jnp.transpose(ry_ref[0]).reshape(1, m),
         jnp.transpose(rz_ref[0]).reshape(1, m)], axis=0)
    h = _bn_relu(jnp.dot(w1[...], x, preferred_element_type=jnp.float32),
                 g1[...], b1[...])
    h = _bn_relu(jnp.dot(w2[...], h, preferred_element_type=jnp.float32),
                 g2[...], b2[...])
    h = _bn_relu(jnp.dot(w3[...], h, preferred_element_type=jnp.float32),
                 g3[...], b3[...])
    out_ref[0] = jnp.max(h.reshape(128, K1, sc), axis=1)


def _sa1_mlp(r1x, r1y, r1z, ws):
    sc = S1 // _SA1_CH
    (w1, g1, b1), (w2, g2, b2), (w3, g3, b3) = ws
    cvec = lambda c: pl.BlockSpec((c, 1), lambda b, ch: (0, 0))
    return pl.pallas_call(
        _sa1_body,
        grid=(B, _SA1_CH),
        in_specs=[
            pl.BlockSpec((1, sc, K1), lambda b, c: (b, c, 0)),
            pl.BlockSpec((1, sc, K1), lambda b, c: (b, c, 0)),
            pl.BlockSpec((1, sc, K1), lambda b, c: (b, c, 0)),
            pl.BlockSpec((64, 3), lambda b, c: (0, 0)), cvec(64), cvec(64),
            pl.BlockSpec((64, 64), lambda b, c: (0, 0)), cvec(64), cvec(64),
            pl.BlockSpec((128, 64), lambda b, c: (0, 0)), cvec(128), cvec(128),
        ],
        out_specs=pl.BlockSpec((1, 128, sc), lambda b, c: (b, 0, c)),
        out_shape=jax.ShapeDtypeStruct((B, 128, S1), jnp.float32),
    )(r1x, r1y, r1z, w1, g1, b1, w2, g2, b2, w3, g3, b3)


_SA2_CH = 8
_SA2_CW = (S2 * K2) // _SA2_CH
_SA2_SPC = _SA2_CW // K2


def _sa2_body(idx_ref, rx_ref, ry_ref, rz_ref, f1_ref,
              w1, g1, b1, w2, g2, b2, w3, g3, b3,
              lw1, lg1, lb1, lw2, lg2, lb2, lw3, lg3, lb3,
              feats_ref, ami_ref):
    f1 = f1_ref[0]
    iota_n = lax.broadcasted_iota(jnp.int32, (S1, _SA2_CW), 0)
    kpc = K2 // _SA2_CH
    idxt = jnp.transpose(idx_ref[0])
    rxt = jnp.transpose(rx_ref[0])
    ryt = jnp.transpose(ry_ref[0])
    rzt = jnp.transpose(rz_ref[0])

    f2 = None
    for c in range(_SA2_CH):
        sl = slice(c * kpc, (c + 1) * kpc)
        idr = idxt[sl].reshape(1, _SA2_CW)
        e = (iota_n == idr).astype(jnp.float32)
        g = jnp.dot(f1, e, preferred_element_type=jnp.float32)
        x = jnp.concatenate(
            [rxt[sl].reshape(1, _SA2_CW),
             ryt[sl].reshape(1, _SA2_CW),
             rzt[sl].reshape(1, _SA2_CW), g], axis=0)
        h = _bn_relu(jnp.dot(w1[...], x, preferred_element_type=jnp.float32),
                     g1[...], b1[...])
        h = _bn_relu(jnp.dot(w2[...], h, preferred_element_type=jnp.float32),
                     g2[...], b2[...])
        h = _bn_relu(jnp.dot(w3[...], h, preferred_element_type=jnp.float32),
                     g3[...], b3[...])
        part = jnp.max(h.reshape(256, kpc, S2), axis=1)
        f2 = part if f2 is None else jnp.maximum(f2, part)
    h = _bn_relu(jnp.dot(lw1[...], f2, preferred_element_type=jnp.float32),
                 lg1[...], lb1[...])
    h = _bn_relu(jnp.dot(lw2[...], h, preferred_element_type=jnp.float32),
                 lg2[...], lb2[...])
    h = _bn_relu(jnp.dot(lw3[...], h, preferred_element_type=jnp.float32),
                 lg3[...], lb3[...])
    mx = jnp.max(h, axis=1)
    feats_ref[0, 0] = mx
    iota_s = lax.broadcasted_iota(jnp.int32, (1024, S2), 1)
    ami = jnp.min(jnp.where(h == mx[:, None], iota_s, S2), axis=1)
    ami_ref[0, 0] = ami.astype(jnp.int32)


def _sa2_local(idx2, r2x, r2y, r2z, feat1, ws2, wsl):
    (w1, g1, b1), (w2, g2, b2), (w3, g3, b3) = ws2
    (lw1, lg1, lb1), (lw2, lg2, lb2), (lw3, lg3, lb3) = wsl
    full = lambda *shape: pl.BlockSpec(shape, lambda b: tuple(0 for _ in shape))
    return pl.pallas_call(
        _sa2_body,
        grid=(B,),
        in_specs=[
            pl.BlockSpec((1, S2, K2), lambda b: (b, 0, 0)),
            pl.BlockSpec((1, S2, K2), lambda b: (b, 0, 0)),
            pl.BlockSpec((1, S2, K2), lambda b: (b, 0, 0)),
            pl.BlockSpec((1, S2, K2), lambda b: (b, 0, 0)),
            pl.BlockSpec((1, 128, S1), lambda b: (b, 0, 0)),
            full(128, 131), full(128, 1), full(128, 1),
            full(128, 128), full(128, 1), full(128, 1),
            full(256, 128), full(256, 1), full(256, 1),
            full(256, 256), full(256, 1), full(256, 1),
            full(512, 256), full(512, 1), full(512, 1),
            full(1024, 512), full(1024, 1), full(1024, 1),
        ],
        out_specs=[
            pl.BlockSpec((1, 1, 1024), lambda b: (b, 0, 0)),
            pl.BlockSpec((1, 1, 1024), lambda b: (b, 0, 0)),
        ],
        out_shape=[
            jax.ShapeDtypeStruct((B, 1, 1024), jnp.float32),
            jax.ShapeDtypeStruct((B, 1, 1024), jnp.int32),
        ],
    )(idx2, r2x, r2y, r2z, feat1, w1, g1, b1, w2, g2, b2, w3, g3, b3,
      lw1, lg1, lb1, lw2, lg2, lb2, lw3, lg3, lb3)


def _prep(layers):
    return [(w, gamma[:, None], beta[:, None]) for w, gamma, beta in layers]


def kernel(points, params):
    cen_pm, pts_pm = _fps(points)
    points_f = pts_pm.reshape(-1)
    centers_f = cen_pm.reshape(-1)

    bq1 = _make_ballquery(N0, S1, K1, R1SQ, want_idx=False)
    r1x, r1y, r1z = bq1(points_f, centers_f)

    sa1 = _prep(params["sa1"])
    feat1 = _sa1_mlp(r1x, r1y, r1z, sa1)

    bq2 = _make_ballquery(S1, S2, K2, R2SQ, want_idx=True)
    r2x, r2y, r2z, idx2 = bq2(centers_f, centers_f)

    ws2 = _prep(params["sa2"])
    wsl = _prep(params["local"])
    feats, ami = _sa2_local(idx2, r2x, r2y, r2z, feat1, ws2, wsl)
    return {"feats": feats[:, 0], "max_indices": ami[:, 0]}


# speedup vs baseline: 2.1434x; 1.4605x over previous
import functools

import jax
import jax.numpy as jnp
from jax import lax
from jax.experimental import pallas as pl
from jax.experimental.pallas import tpu as pltpu
from jax.experimental.pallas import tpu_sc as plsc

B = 16
N0 = 2048
S1 = 512
S2 = 128
K1 = 32
K2 = 64
R1SQ = 0.2 * 0.2
R2SQ = 0.4 * 0.4
EPS = 1e-5

NC = 2
NS = 16
LANES = 16
NTILES = NC * NS


def _fps_body(pts_ref, cen_ref, pp_ref, sx, sy, sz):
    x = pts_ref[:, 0, :]
    y = pts_ref[:, 1, :]
    z = pts_ref[:, 2, :]
    iota_n = lax.broadcasted_iota(jnp.int32, (B, N0), 1)

    def step(t, carry):
        dist, idxv = carry
        oh = (iota_n == idxv).astype(jnp.float32)
        cx = jnp.sum(x * oh, axis=1, keepdims=True)
        cy = jnp.sum(y * oh, axis=1, keepdims=True)
        cz = jnp.sum(z * oh, axis=1, keepdims=True)
        sx[pl.ds(t, 1), :] = cx.reshape(1, B)
        sy[pl.ds(t, 1), :] = cy.reshape(1, B)
        sz[pl.ds(t, 1), :] = cz.reshape(1, B)
        dx = x - cx
        dy = y - cy
        dz = z - cz
        d = (dx * dx + dy * dy) + dz * dz
        dist = jnp.minimum(dist, d)
        mx = jnp.max(dist, axis=1, keepdims=True)
        idxv = jnp.min(jnp.where(dist == mx, iota_n, N0), axis=1, keepdims=True)
        return dist, idxv

    dist0 = jnp.full((B, N0), 1e10, dtype=jnp.float32)
    idx0 = jnp.zeros((B, 1), dtype=jnp.int32)
    lax.fori_loop(0, S1, step, (dist0, idx0))
    for c, sref in enumerate((sx, sy, sz)):
        cen_ref[pl.ds(c * B, B), :] = jnp.transpose(sref[...])
        pp_ref[pl.ds(c * B, B), :] = pts_ref[:, c, :]


def _fps(points):
    return pl.pallas_call(
        _fps_body,
        out_shape=[
            jax.ShapeDtypeStruct((3 * B, S1), jnp.float32),
            jax.ShapeDtypeStruct((3 * B, N0), jnp.float32),
        ],
        scratch_shapes=[pltpu.VMEM((S1, B), jnp.float32),
                        pltpu.VMEM((S1, B), jnp.float32),
                        pltpu.VMEM((S1, B), jnp.float32)],
    )(points)


def _make_ballquery(n, s, k, r2, want_idx):
    spt = (B * s) // NTILES
    tiles_per_b = s // spt
    nchunks = n // LANES
    UB = 16
    nblocks = nchunks // UB
    bufsz = k + LANES

    def body(*refs):
        pts_hbm, cen_hbm = refs[0], refs[1]
        if want_idx:
            outs = refs[2:6]
            vpx, vpy, vpz, vcx, vcy, vcz, vbuf, vrel, vidx, sem = refs[6:]
        else:
            outs = refs[2:5]
            vpx, vpy, vpz, vcx, vcy, vcz, vbuf, vrel, sem = refs[5:]
            vidx = None
        w = lax.axis_index("s") * NC + lax.axis_index("c")
        b = w // tiles_per_b
        s0 = (w % tiles_per_b) * spt
        pltpu.sync_copy(pts_hbm.at[pl.ds(b * n, n)], vpx)
        pltpu.sync_copy(pts_hbm.at[pl.ds((B + b) * n, n)], vpy)
        pltpu.sync_copy(pts_hbm.at[pl.ds((2 * B + b) * n, n)], vpz)
        cbase = b * S1 + s0
        pltpu.sync_copy(cen_hbm.at[pl.ds(cbase, spt)], vcx)
        pltpu.sync_copy(cen_hbm.at[pl.ds(cbase + B * S1, spt)], vcy)
        pltpu.sync_copy(cen_hbm.at[pl.ds(cbase + 2 * B * S1, spt)], vcz)
        lane = lax.iota(jnp.int32, LANES)
        zero = lane * 0

        def per_s(si, _):
            sv = zero + si
            cx = plsc.load_gather(vcx, [sv])
            cy = plsc.load_gather(vcy, [sv])
            cz = plsc.load_gather(vcz, [sv])

            def cond(c):
                i, cnt = c
                return jnp.logical_and(i < nblocks, cnt < k)

            def block(c):
                i, cnt = c
                base = i * (UB * LANES)
                ms, mis, incls = [], [], []
                for u in range(UB):
                    ii = base + u * LANES
                    dx = vpx[pl.ds(ii, LANES)] - cx
                    dy = vpy[pl.ds(ii, LANES)] - cy
                    dz = vpz[pl.ds(ii, LANES)] - cz
                    d2 = (dx * dx + dy * dy) + dz * dz
                    m = d2 < r2
                    mi = m.astype(jnp.int32)
                    ms.append(m)
                    mis.append(mi)
                    incls.append(plsc.cumsum(mi))
                bases = []
                for u in range(UB):
                    bases.append(cnt)
                    cnt = cnt + incls[u][LANES - 1]
                for u in range(UB):
                    ii = base + u * LANES
                    pos = jnp.minimum(bases[u], k) + (incls[u] - mis[u])
                    plsc.store_scatter(vbuf, [pos], ii + lane, mask=ms[u])
                return i + 1, cnt

            _, cnt = lax.while_loop(cond, block, (0, 0))
            valid = jnp.minimum(cnt, k)
            for j in range(k // LANES):
                p = j * LANES + lane
                pos = jnp.where(p < valid, p, 0)
                gi = plsc.load_gather(vbuf, [pos])
                vrel[0, si, pl.ds(j * LANES, LANES)] = (
                    plsc.load_gather(vpx, [gi]) - cx)
                vrel[1, si, pl.ds(j * LANES, LANES)] = (
                    plsc.load_gather(vpy, [gi]) - cy)
                vrel[2, si, pl.ds(j * LANES, LANES)] = (
                    plsc.load_gather(vpz, [gi]) - cz)
                if want_idx:
                    vidx[si, pl.ds(j * LANES, LANES)] = gi
            return _

        lax.fori_loop(0, spt, per_s, 0)
        pltpu.sync_copy(vrel.at[0], outs[0].at[b, pl.ds(s0, spt), :])
        pltpu.sync_copy(vrel.at[1], outs[1].at[b, pl.ds(s0, spt), :])
        pltpu.sync_copy(vrel.at[2], outs[2].at[b, pl.ds(s0, spt), :])
        if want_idx:
            pltpu.sync_copy(vidx, outs[3].at[b, pl.ds(s0, spt), :])

    mesh = plsc.VectorSubcoreMesh(core_axis_name="c", subcore_axis_name="s")
    out_type = [jax.ShapeDtypeStruct((B, s, k), jnp.float32)] * 3
    if want_idx:
        out_type.append(jax.ShapeDtypeStruct((B, s, k), jnp.int32))
    scratch = [
        pltpu.VMEM((n,), jnp.float32),
        pltpu.VMEM((n,), jnp.float32),
        pltpu.VMEM((n,), jnp.float32),
        pltpu.VMEM((spt,), jnp.float32),
        pltpu.VMEM((spt,), jnp.float32),
        pltpu.VMEM((spt,), jnp.float32),
        pltpu.VMEM((bufsz,), jnp.int32),
        pltpu.VMEM((3, spt, k), jnp.float32),
    ]
    if want_idx:
        scratch.append(pltpu.VMEM((spt, k), jnp.int32))
    scratch.append(pltpu.SemaphoreType.DMA)
    return pl.kernel(body, out_type=tuple(out_type), mesh=mesh,
                     scratch_types=scratch,
                     compiler_params=pltpu.CompilerParams(
                         needs_layout_passes=False))


_SA1_CH = 4


def _bn_relu(y, gamma, beta):
    return jnp.maximum((y / jnp.sqrt(1.0 + EPS)) * gamma + beta, 0.0)


def _sa1_body(rx_ref, ry_ref, rz_ref, w1, g1, b1, w2, g2, b2, w3, g3, b3,
              out_ref):
    sc = S1 // _SA1_CH
    m = sc * K1
    x = jnp.concatenate(
        [jnp.transpose(rx_ref[0]).reshape(1, m),
         jnp.transpose(ry_ref[0]).reshape(1, m),
         jnp.transpose(rz_ref[0]).reshape(1, m)], axis=0)
    h = _bn_relu(jnp.dot(w1[...], x, preferred_element_type=jnp.float32),
                 g1[...], b1[...])
    h = _bn_relu(jnp.dot(w2[...], h, preferred_element_type=jnp.float32),
                 g2[...], b2[...])
    h = _bn_relu(jnp.dot(w3[...], h, preferred_element_type=jnp.float32),
                 g3[...], b3[...])
    out_ref[0] = jnp.max(h.reshape(128, K1, sc), axis=1)


def _sa1_mlp(r1x, r1y, r1z, ws):
    sc = S1 // _SA1_CH
    (w1, g1, b1), (w2, g2, b2), (w3, g3, b3) = ws
    cvec = lambda c: pl.BlockSpec((c, 1), lambda b, ch: (0, 0))
    return pl.pallas_call(
        _sa1_body,
        grid=(B, _SA1_CH),
        in_specs=[
            pl.BlockSpec((1, sc, K1), lambda b, c: (b, c, 0)),
            pl.BlockSpec((1, sc, K1), lambda b, c: (b, c, 0)),
            pl.BlockSpec((1, sc, K1), lambda b, c: (b, c, 0)),
            pl.BlockSpec((64, 3), lambda b, c: (0, 0)), cvec(64), cvec(64),
            pl.BlockSpec((64, 64), lambda b, c: (0, 0)), cvec(64), cvec(64),
            pl.BlockSpec((128, 64), lambda b, c: (0, 0)), cvec(128), cvec(128),
        ],
        out_specs=pl.BlockSpec((1, 128, sc), lambda b, c: (b, 0, c)),
        out_shape=jax.ShapeDtypeStruct((B, 128, S1), jnp.float32),
    )(r1x, r1y, r1z, w1, g1, b1, w2, g2, b2, w3, g3, b3)


_SA2_CH = 8
_SA2_CW = (S2 * K2) // _SA2_CH
_SA2_SPC = _SA2_CW // K2


def _sa2_body(idx_ref, rx_ref, ry_ref, rz_ref, f1_ref,
              w1, g1, b1, w2, g2, b2, w3, g3, b3,
              lw1, lg1, lb1, lw2, lg2, lb2, lw3, lg3, lb3,
              feats_ref, ami_ref):
    f1 = f1_ref[0]
    iota_n = lax.broadcasted_iota(jnp.int32, (S1, _SA2_CW), 0)
    kpc = K2 // _SA2_CH
    idxt = jnp.transpose(idx_ref[0])
    rxt = jnp.transpose(rx_ref[0])
    ryt = jnp.transpose(ry_ref[0])
    rzt = jnp.transpose(rz_ref[0])

    f2 = None
    for c in range(_SA2_CH):
        sl = slice(c * kpc, (c + 1) * kpc)
        idr = idxt[sl].reshape(1, _SA2_CW)
        e = (iota_n == idr).astype(jnp.float32)
        g = jnp.dot(f1, e, preferred_element_type=jnp.float32)
        x = jnp.concatenate(
            [rxt[sl].reshape(1, _SA2_CW),
             ryt[sl].reshape(1, _SA2_CW),
             rzt[sl].reshape(1, _SA2_CW), g], axis=0)
        h = _bn_relu(jnp.dot(w1[...], x, preferred_element_type=jnp.float32),
                     g1[...], b1[...])
        h = _bn_relu(jnp.dot(w2[...], h, preferred_element_type=jnp.float32),
                     g2[...], b2[...])
        h = _bn_relu(jnp.dot(w3[...], h, preferred_element_type=jnp.float32),
                     g3[...], b3[...])
        part = jnp.max(h.reshape(256, kpc, S2), axis=1)
        f2 = part if f2 is None else jnp.maximum(f2, part)
    h = _bn_relu(jnp.dot(lw1[...], f2, preferred_element_type=jnp.float32),
                 lg1[...], lb1[...])
    h = _bn_relu(jnp.dot(lw2[...], h, preferred_element_type=jnp.float32),
                 lg2[...], lb2[...])
    h = _bn_relu(jnp.dot(lw3[...], h, preferred_element_type=jnp.float32),
                 lg3[...], lb3[...])
    mx = jnp.max(h, axis=1)
    feats_ref[0, 0] = mx
    iota_s = lax.broadcasted_iota(jnp.int32, (1024, S2), 1)
    ami = jnp.min(jnp.where(h == mx[:, None], iota_s, S2), axis=1)
    ami_ref[0, 0] = ami.astype(jnp.int32)


def _sa2_local(idx2, r2x, r2y, r2z, feat1, ws2, wsl):
    (w1, g1, b1), (w2, g2, b2), (w3, g3, b3) = ws2
    (lw1, lg1, lb1), (lw2, lg2, lb2), (lw3, lg3, lb3) = wsl
    full = lambda *shape: pl.BlockSpec(shape, lambda b: tuple(0 for _ in shape))
    return pl.pallas_call(
        _sa2_body,
        grid=(B,),
        in_specs=[
            pl.BlockSpec((1, S2, K2), lambda b: (b, 0, 0)),
            pl.BlockSpec((1, S2, K2), lambda b: (b, 0, 0)),
            pl.BlockSpec((1, S2, K2), lambda b: (b, 0, 0)),
            pl.BlockSpec((1, S2, K2), lambda b: (b, 0, 0)),
            pl.BlockSpec((1, 128, S1), lambda b: (b, 0, 0)),
            full(128, 131), full(128, 1), full(128, 1),
            full(128, 128), full(128, 1), full(128, 1),
            full(256, 128), full(256, 1), full(256, 1),
            full(256, 256), full(256, 1), full(256, 1),
            full(512, 256), full(512, 1), full(512, 1),
            full(1024, 512), full(1024, 1), full(1024, 1),
        ],
        out_specs=[
            pl.BlockSpec((1, 1, 1024), lambda b: (b, 0, 0)),
            pl.BlockSpec((1, 1, 1024), lambda b: (b, 0, 0)),
        ],
        out_shape=[
            jax.ShapeDtypeStruct((B, 1, 1024), jnp.float32),
            jax.ShapeDtypeStruct((B, 1, 1024), jnp.int32),
        ],
    )(idx2, r2x, r2y, r2z, feat1, w1, g1, b1, w2, g2, b2, w3, g3, b3,
      lw1, lg1, lb1, lw2, lg2, lb2, lw3, lg3, lb3)


def _prep(layers):
    return [(w, gamma[:, None], beta[:, None]) for w, gamma, beta in layers]


def kernel(points, params):
    cen_pm, pts_pm = _fps(points)
    points_f = pts_pm.reshape(-1)
    centers_f = cen_pm.reshape(-1)

    bq1 = _make_ballquery(N0, S1, K1, R1SQ, want_idx=False)
    r1x, r1y, r1z = bq1(points_f, centers_f)

    sa1 = _prep(params["sa1"])
    feat1 = _sa1_mlp(r1x, r1y, r1z, sa1)

    bq2 = _make_ballquery(S1, S2, K2, R2SQ, want_idx=True)
    r2x, r2y, r2z, idx2 = bq2(centers_f, centers_f)

    ws2 = _prep(params["sa2"])
    wsl = _prep(params["local"])
    feats, ami = _sa2_local(idx2, r2x, r2y, r2z, feat1, ws2, wsl)
    return {"feats": feats[:, 0], "max_indices": ami[:, 0]}
